# baseline (speedup 1.0000x reference)
import functools

import jax
import jax.numpy as jnp
from jax import lax
from jax.experimental import pallas as pl
from jax.experimental.pallas import tpu as pltpu
from jax.experimental.pallas import tpu_sc as plsc

_NUM_BOXES = 100000
_DIMS = 64
_BATCH = 16384
_ROW = 2 * _DIMS
_NW = 32
_CHUNK = 128
_NCHUNK = _BATCH // _NW // _CHUNK
_TCOLS = _BATCH // _CHUNK


def _make_kernel():
    mesh = plsc.VectorSubcoreMesh(core_axis_name="c", subcore_axis_name="s")

    @functools.partial(
        pl.kernel,
        mesh=mesh,
        out_type=jax.ShapeDtypeStruct((2 * _DIMS * _BATCH,), jnp.float32),
        scratch_types=[
            pltpu.VMEM((_NCHUNK, _CHUNK), jnp.int32),
            pltpu.VMEM((3, _CHUNK, _ROW), jnp.float32),
            pltpu.VMEM((_ROW * _CHUNK,), jnp.float32),
            pltpu.VMEM((_ROW * _CHUNK,), jnp.float32),
            pltpu.SemaphoreType.DMA((3,)),
            pltpu.SemaphoreType.DMA((2,)),
        ],
        compiler_params=pltpu.CompilerParams(needs_layout_passes=False),
    )
    def gather_kernel(idx_hbm, tbl_hbm, out_hbm, idx_v, rows_v, tr_v0,
                      tr_v1, sem_g, sem_o):
        i32 = jnp.int32
        iota = lax.iota(i32, 16)
        wid = lax.axis_index("s") * 2 + lax.axis_index("c")

        def splat(x):
            return jnp.full((16,), x, i32)

        pltpu.sync_copy(idx_hbm.at[pl.ds(wid * _NCHUNK, _NCHUNK)], idx_v)

        def fire_gather(j):
            return pltpu.async_copy(
                tbl_hbm.at[idx_v.at[j]], rows_v.at[j % 3], sem_g.at[j % 3])

        gather_descs = {0: fire_gather(0), 1: fire_gather(1),
                        2: fire_gather(2)}
        out_descs = {}

        lane_vecs = [splat(k * 16) + iota for k in range(_ROW // 16)]
        tr_bases = [v << 7 for v in lane_vecs]

        for j in range(_NCHUNK):
            rows_ref = rows_v.at[j % 3]
            tr_ref = (tr_v0, tr_v1)[j % 2]
            gather_descs[j].wait()
            if j >= 2:
                for d in out_descs.pop(j - 2):
                    d.wait()

            def transpose_row(r, rows_ref=rows_ref, tr_ref=tr_ref):
                r_spl = splat(r)
                for k in range(_ROW // 16):
                    vals = plsc.load_gather(rows_ref, [r_spl, lane_vecs[k]])
                    plsc.store_scatter(tr_ref, [tr_bases[k] + r_spl], vals)
                return None

            pl.loop(0, _CHUNK, unroll=4)(transpose_row)

            t_out = wid * _NCHUNK + j
            descs = []
            for c in range(2 * _DIMS // 8):
                descs.append(pltpu.async_copy(
                    tr_ref.at[pl.ds(c * 8 * _CHUNK, 8 * _CHUNK)],
                    out_hbm.at[pl.ds((c * _TCOLS + t_out) * 8 * _CHUNK,
                                     8 * _CHUNK)],
                    sem_o.at[j % 2]))
            out_descs[j] = descs

            if j + 3 < _NCHUNK:
                gather_descs[j + 3] = fire_gather(j + 3)

        for j in (_NCHUNK - 2, _NCHUNK - 1):
            for d in out_descs.pop(j):
                d.wait()

    return gather_kernel


_gather = _make_kernel()


def kernel(box_indices, boxes):
    idx = box_indices.astype(jnp.int32).reshape(_TCOLS, _CHUNK)
    tbl = boxes.reshape(_NUM_BOXES, _ROW)
    out5 = _gather(idx, tbl).reshape(2, _DIMS // 8, _TCOLS, 8, _CHUNK)
    out = out5.transpose(2, 4, 0, 1, 3).reshape(_BATCH, 2, _DIMS)
    return out[None]


# speedup vs baseline: 1.2158x; 1.2158x over previous
import functools

import jax
import jax.numpy as jnp
from jax import lax
from jax.experimental import pallas as pl
from jax.experimental.pallas import tpu as pltpu
from jax.experimental.pallas import tpu_sc as plsc

_NUM_BOXES = 100000
_DIMS = 64
_BATCH = 16384
_ROW = 2 * _DIMS

_NC = 2
_NS = 16
_NW = _NC * _NS
_B_PER_W = _BATCH // _NW
_CHUNK = 128
_NCHUNK = _B_PER_W // _CHUNK


def _make_gather():
    mesh = plsc.VectorSubcoreMesh(core_axis_name="c", subcore_axis_name="s")

    @functools.partial(
        pl.kernel,
        mesh=mesh,
        out_type=jax.ShapeDtypeStruct((_NW * _NCHUNK, _CHUNK, _ROW),
                                      jnp.float32),
        scratch_types=[
            pltpu.VMEM((_NCHUNK, _CHUNK), jnp.int32),
            pltpu.VMEM((_NCHUNK, _CHUNK, _ROW), jnp.float32),
            pltpu.SemaphoreType.DMA,
            pltpu.SemaphoreType.DMA,
        ],
    )
    def gather_kernel(idx_hbm, tbl_hbm, out_hbm, idx_v, rows_v, sem_g,
                      sem_o):
        wid = lax.axis_index("s") * _NC + lax.axis_index("c")
        pltpu.sync_copy(idx_hbm.at[pl.ds(wid * _NCHUNK, _NCHUNK)], idx_v)
        gathers = [
            pltpu.async_copy(tbl_hbm.at[idx_v.at[j]], rows_v.at[j], sem_g)
            for j in range(_NCHUNK)
        ]
        stores = []
        for j in range(_NCHUNK):
            gathers[j].wait()
            stores.append(pltpu.async_copy(
                rows_v.at[j], out_hbm.at[wid * _NCHUNK + j], sem_o))
        for s in stores:
            s.wait()

    return gather_kernel


_gather = _make_gather()


def kernel(box_indices, boxes):
    idx = box_indices.astype(jnp.int32).reshape(_NW * _NCHUNK, _CHUNK)
    table = boxes.reshape(_NUM_BOXES, _ROW)
    out = _gather(idx, table)
    return out.reshape(1, _BATCH, 2, _DIMS)


# speedup vs baseline: 1.2247x; 1.0073x over previous
import functools

import jax
import jax.numpy as jnp
from jax import lax
from jax.experimental import pallas as pl
from jax.experimental.pallas import tpu as pltpu
from jax.experimental.pallas import tpu_sc as plsc

_NUM_BOXES = 100000
_DIMS = 64
_BATCH = 16384
_ROW = 2 * _DIMS

_NC = 2
_NS = 16
_NW = _NC * _NS
_B_PER_W = _BATCH // _NW
_CHUNK = 128
_NCHUNK = _B_PER_W // _CHUNK


def _make_gather():
    mesh = plsc.VectorSubcoreMesh(core_axis_name="c", subcore_axis_name="s")

    @functools.partial(
        pl.kernel,
        mesh=mesh,
        out_type=jax.ShapeDtypeStruct((_NW * _NCHUNK, _CHUNK, _ROW), jnp.float32),
        scratch_types=[
            pltpu.VMEM((_NCHUNK, _CHUNK), jnp.int32),
            pltpu.VMEM((_NCHUNK, _CHUNK, _ROW), jnp.float32),
            pltpu.SemaphoreType.DMA,
        ],
    )
    def gather_kernel(idx_hbm, table_hbm, out_hbm, idx_v, rows_v, sem):
        wid = lax.axis_index("s") * _NC + lax.axis_index("c")
        pltpu.sync_copy(idx_hbm.at[pl.ds(wid * _NCHUNK, _NCHUNK)], idx_v)
        copies = []
        for j in range(_NCHUNK):
            copies.append(
                pltpu.async_copy(table_hbm.at[idx_v.at[j]], rows_v.at[j], sem)
            )
        for c in copies:
            c.wait()
        pltpu.sync_copy(rows_v, out_hbm.at[pl.ds(wid * _NCHUNK, _NCHUNK)])

    return gather_kernel


_gather = _make_gather()


def kernel(box_indices, boxes):
    idx = box_indices.astype(jnp.int32).reshape(_NW * _NCHUNK, _CHUNK)
    table = boxes.reshape(_NUM_BOXES, _ROW)
    out = _gather(idx, table)
    return out.reshape(1, _BATCH, 2, _DIMS)
